# baseline (speedup 1.0000x reference)
import jax
import jax.numpy as jnp
from jax.experimental import pallas as pl
from jax.experimental.pallas import tpu as pltpu

_TRANS_RHS = (((1,), (1,)), ((), ()))


def _round_up(x, m):
    return ((x + m - 1) // m) * m


def _fused_kernel(x_ref, w1_ref, b1_ref, w2_ref, b2_ref, adj_ref,
                  ha32_ref, hp_ref, ha16_ref, ha32_s_ref):
    i = pl.program_id(0)
    tm = adj_ref.shape[0]

    @pl.when(i == 0)
    def _():
        h = jax.lax.dot_general(x_ref[...], w1_ref[...], _TRANS_RHS,
                                preferred_element_type=jnp.float32)
        h = jnp.maximum(h + b1_ref[...][None, :], 0.0)
        ha = jax.lax.dot_general(h, w2_ref[...], _TRANS_RHS,
                                 preferred_element_type=jnp.float32)
        ha = ha + b2_ref[...][None, :]
        ha32_s_ref[...] = ha
        ha16_ref[...] = ha.astype(jnp.bfloat16)

    ha32_ref[...] = ha32_s_ref[pl.ds(i * tm, tm), :]
    a16 = adj_ref[...].astype(jnp.bfloat16)
    hp_ref[...] = jnp.dot(a16, ha16_ref[...],
                          preferred_element_type=jnp.float32)


def kernel(x, adj, w1, b1, w2, b2):
    N, n_in = x.shape
    h1 = w1.shape[0]
    h2 = w2.shape[0]
    f32 = jnp.float32

    TM = 512 if N % 512 == 0 else 256
    N_pad = _round_up(N, TM)

    x_p = x.astype(f32)
    adj_p = adj.astype(f32)
    if N_pad != N:
        x_p = jnp.pad(x_p, ((0, N_pad - N), (0, 0)))
        adj_p = jnp.pad(adj_p, ((0, N_pad - N), (0, N_pad - N)))

    ha32, hp = pl.pallas_call(
        _fused_kernel,
        out_shape=(
            jax.ShapeDtypeStruct((N_pad, h2), f32),
            jax.ShapeDtypeStruct((N_pad, h2), f32),
        ),
        grid=(N_pad // TM,),
        in_specs=[
            pl.BlockSpec((N_pad, n_in), lambda i: (0, 0)),
            pl.BlockSpec((h1, n_in), lambda i: (0, 0)),
            pl.BlockSpec((h1,), lambda i: (0,)),
            pl.BlockSpec((h2, h1), lambda i: (0, 0)),
            pl.BlockSpec((h2,), lambda i: (0,)),
            pl.BlockSpec((TM, N_pad), lambda i: (i, 0)),
        ],
        out_specs=(
            pl.BlockSpec((TM, h2), lambda i: (i, 0)),
            pl.BlockSpec((TM, h2), lambda i: (i, 0)),
        ),
        scratch_shapes=[pltpu.VMEM((N_pad, h2), jnp.bfloat16),
                        pltpu.VMEM((N_pad, h2), jnp.float32)],
        compiler_params=pltpu.CompilerParams(
            dimension_semantics=("arbitrary",)),
        cost_estimate=pl.CostEstimate(
            flops=2 * N_pad * N_pad * h2
                  + 2 * N_pad * n_in * h1 + 2 * N_pad * h1 * h2,
            transcendentals=0,
            bytes_accessed=4 * N_pad * N_pad + 4 * N_pad * n_in
                           + 8 * N_pad * h2,
        ),
    )(x_p, w1, b1, w2, b2, adj_p)

    return ha32[:N, :h2], hp[:N, :h2]


# speedup vs baseline: 1.0246x; 1.0246x over previous
import jax
import jax.numpy as jnp
from jax.experimental import pallas as pl
from jax.experimental.pallas import tpu as pltpu

_TRANS_RHS = (((1,), (1,)), ((), ()))


def _round_up(x, m):
    return ((x + m - 1) // m) * m


def _fused_kernel(x_ref, w1_ref, b1_ref, w2_ref, b2_ref, adj_ref,
                  ha32_ref, hp_ref, ha16_ref):
    i = pl.program_id(0)
    tm = adj_ref.shape[0]

    @pl.when(i == 0)
    def _():
        h = jax.lax.dot_general(x_ref[...], w1_ref[...], _TRANS_RHS,
                                preferred_element_type=jnp.float32)
        h = jnp.maximum(h + b1_ref[...][None, :], 0.0)
        ha = jax.lax.dot_general(h, w2_ref[...], _TRANS_RHS,
                                 preferred_element_type=jnp.float32)
        ha = ha + b2_ref[...][None, :]
        ha32_ref[...] = ha
        ha16_ref[...] = ha.astype(jnp.bfloat16)

    a16 = adj_ref[...].astype(jnp.bfloat16)
    hp_ref[pl.ds(i * tm, tm), :] = jnp.dot(
        a16, ha16_ref[...], preferred_element_type=jnp.float32)


def kernel(x, adj, w1, b1, w2, b2):
    N, n_in = x.shape
    h1 = w1.shape[0]
    h2 = w2.shape[0]
    f32 = jnp.float32

    TM = 512 if N % 512 == 0 else 256
    N_pad = _round_up(N, TM)

    x_p = x.astype(f32)
    adj_p = adj.astype(f32)
    if N_pad != N:
        x_p = jnp.pad(x_p, ((0, N_pad - N), (0, 0)))
        adj_p = jnp.pad(adj_p, ((0, N_pad - N), (0, N_pad - N)))

    ha32, hp = pl.pallas_call(
        _fused_kernel,
        out_shape=(
            jax.ShapeDtypeStruct((N_pad, h2), f32),
            jax.ShapeDtypeStruct((N_pad, h2), f32),
        ),
        grid=(N_pad // TM,),
        in_specs=[
            pl.BlockSpec((N_pad, n_in), lambda i: (0, 0)),
            pl.BlockSpec((h1, n_in), lambda i: (0, 0)),
            pl.BlockSpec((h1,), lambda i: (0,)),
            pl.BlockSpec((h2, h1), lambda i: (0, 0)),
            pl.BlockSpec((h2,), lambda i: (0,)),
            pl.BlockSpec((TM, N_pad), lambda i: (i, 0)),
        ],
        out_specs=(
            pl.BlockSpec((N_pad, h2), lambda i: (0, 0)),
            pl.BlockSpec((N_pad, h2), lambda i: (0, 0)),
        ),
        scratch_shapes=[pltpu.VMEM((N_pad, h2), jnp.bfloat16)],
        compiler_params=pltpu.CompilerParams(
            dimension_semantics=("arbitrary",)),
        cost_estimate=pl.CostEstimate(
            flops=2 * N_pad * N_pad * h2
                  + 2 * N_pad * n_in * h1 + 2 * N_pad * h1 * h2,
            transcendentals=0,
            bytes_accessed=4 * N_pad * N_pad + 4 * N_pad * n_in
                           + 8 * N_pad * h2,
        ),
    )(x_p, w1, b1, w2, b2, adj_p)

    return ha32[:N, :h2], hp[:N, :h2]
